# Initial kernel scaffold; baseline (speedup 1.0000x reference)
import jax
import jax.numpy as jnp
from jax.experimental import pallas as pl


def kernel(x, pers, batch, W1, b1, G1W, G1b, L1W, G2W, G2b, L2W, bn_g, bn_b):
    raise NotImplementedError("write your pallas kernel here")


# speedup vs baseline: 1.0512x; 1.0512x over previous
import jax
import jax.numpy as jnp
from jax.experimental import pallas as pl
from jax.experimental.pallas import tpu as pltpu

NUM_SEGMENTS = 512
BN_EPS = 1e-5
_PREC = jax.lax.Precision.HIGHEST


def _onehot_t(b_row, n_rows):
    ids = jax.lax.broadcasted_iota(jnp.int32, (NUM_SEGMENTS, n_rows), 0)
    return (ids == b_row).astype(jnp.float32)


def _dot(a, b):
    return jax.lax.dot_general(a, b, (((1,), (0,)), ((), ())),
                               precision=_PREC, preferred_element_type=jnp.float32)


def _dot_t(a, b):
    return jax.lax.dot_general(a, b, (((0,), (0,)), ((), ())),
                               precision=_PREC, preferred_element_type=jnp.float32)


def _pass_a(pers_ref, b3_ref, w1_ref, b1_ref, seg1_ref):
    i = pl.program_id(0)
    br = pers_ref.shape[0]
    x0 = jnp.maximum(_dot(pers_ref[...], w1_ref[...]) + b1_ref[...], 0.0)
    ot = _onehot_t(b3_ref[0], br)
    x0e = jnp.concatenate([x0, jnp.ones((br, 1), jnp.float32)], axis=1)

    @pl.when(i == 0)
    def _():
        seg1_ref[...] = jnp.zeros_like(seg1_ref)

    seg1_ref[...] += _dot(ot, x0e)


def _pass_b(pers_ref, b3_ref, seg1_ref, w1_ref, b1_ref, g1w_ref, g1b_ref,
            l1w_ref, h1_ref, seg2_ref, c1_ref):
    i = pl.program_id(0)
    br = pers_ref.shape[0]

    @pl.when(i == 0)
    def _():
        s = seg1_ref[...]
        cnt = jnp.clip(s[:, 64:65], 1.0, None)
        c1_ref[...] = _dot(s[:, :64] / cnt, l1w_ref[...])
        seg2_ref[...] = jnp.zeros_like(seg2_ref)

    x0 = jnp.maximum(_dot(pers_ref[...], w1_ref[...]) + b1_ref[...], 0.0)
    ot = _onehot_t(b3_ref[0], br)
    g = _dot_t(ot, c1_ref[...])
    h1 = jnp.maximum(_dot(x0, g1w_ref[...]) + g1b_ref[...] - g, 0.0)
    h1_ref[...] = h1
    seg2_ref[...] += _dot(ot, h1)


def _pass_c(h1_ref, b3_ref, seg1_ref, seg2_ref, g2w_ref, g2b_ref, l2w_ref,
            h_ref, stats_ref, c2_ref):
    i = pl.program_id(0)
    br = h1_ref.shape[0]

    @pl.when(i == 0)
    def _():
        cnt = jnp.clip(seg1_ref[:, 64:65], 1.0, None)
        c2_ref[...] = _dot(seg2_ref[...] / cnt, l2w_ref[...])
        stats_ref[...] = jnp.zeros_like(stats_ref)

    ot = _onehot_t(b3_ref[0], br)
    g2 = _dot_t(ot, c2_ref[...])
    h = jnp.maximum(_dot(h1_ref[...], g2w_ref[...]) + g2b_ref[...] - g2, 0.0)
    h_ref[...] = h
    s1 = jnp.sum(h, axis=0, keepdims=True)
    s2 = jnp.sum(h * h, axis=0, keepdims=True)
    stats_ref[...] += jnp.concatenate([s1, s2], axis=0)


def _pass_d(x_ref, h_ref, stats_ref, bng_ref, bnb_ref, out_ref, ss_ref, *, n_rows):
    i = pl.program_id(0)

    @pl.when(i == 0)
    def _():
        mu = stats_ref[0:1] * (1.0 / n_rows)
        ex2 = stats_ref[1:2] * (1.0 / n_rows)
        var = jnp.maximum(ex2 - mu * mu, 0.0)
        scale = jax.lax.rsqrt(var + BN_EPS) * bng_ref[...]
        shift = bnb_ref[...] - mu * scale
        ss_ref[...] = jnp.concatenate([scale, shift], axis=0)

    out_ref[...] = x_ref[...] + h_ref[...] * ss_ref[0:1] + ss_ref[1:2]


def kernel(x, pers, batch, W1, b1, G1W, G1b, L1W, G2W, G2b, L2W, bn_g, bn_b):
    n, f = x.shape
    p = pers.shape[1]
    d = W1.shape[1]
    br = 2000 if n % 2000 == 0 else n
    nb = n // br
    batch = batch.astype(jnp.int32)
    b3 = batch.reshape(nb, 1, br)
    b1r, g1br, g2br = b1.reshape(1, d), G1b.reshape(1, d), G2b.reshape(1, f)
    bngr, bnbr = bn_g.reshape(1, f), bn_b.reshape(1, f)

    row_spec = lambda w: pl.BlockSpec((br, w), lambda i: (i, 0))
    b3_spec = pl.BlockSpec((1, 1, br), lambda i: (i, 0, 0))
    const = lambda shape: pl.BlockSpec(shape, lambda i: tuple(0 for _ in shape))
    f32 = jnp.float32

    seg1 = pl.pallas_call(
        _pass_a,
        grid=(nb,),
        in_specs=[row_spec(p), b3_spec, const((p, d)), const((1, d))],
        out_specs=const((NUM_SEGMENTS, d + 1)),
        out_shape=jax.ShapeDtypeStruct((NUM_SEGMENTS, d + 1), f32),
    )(pers, b3, W1, b1r)

    h1, seg2 = pl.pallas_call(
        _pass_b,
        grid=(nb,),
        in_specs=[row_spec(p), b3_spec, const((NUM_SEGMENTS, d + 1)),
                  const((p, d)), const((1, d)), const((d, d)), const((1, d)),
                  const((d, d))],
        out_specs=[row_spec(d), const((NUM_SEGMENTS, d))],
        out_shape=[jax.ShapeDtypeStruct((n, d), f32),
                   jax.ShapeDtypeStruct((NUM_SEGMENTS, d), f32)],
        scratch_shapes=[pltpu.VMEM((NUM_SEGMENTS, d), f32)],
    )(pers, b3, seg1, W1, b1r, G1W, g1br, L1W)

    h, stats = pl.pallas_call(
        _pass_c,
        grid=(nb,),
        in_specs=[row_spec(d), b3_spec, const((NUM_SEGMENTS, d + 1)),
                  const((NUM_SEGMENTS, d)), const((d, f)), const((1, f)),
                  const((d, f))],
        out_specs=[row_spec(f), const((2, f))],
        out_shape=[jax.ShapeDtypeStruct((n, f), f32),
                   jax.ShapeDtypeStruct((2, f), f32)],
        scratch_shapes=[pltpu.VMEM((NUM_SEGMENTS, f), f32)],
    )(h1, b3, seg1, seg2, G2W, g2br, L2W)

    import functools as _ft
    out = pl.pallas_call(
        _ft.partial(_pass_d, n_rows=n),
        grid=(nb,),
        in_specs=[row_spec(f), row_spec(f), const((2, f)), const((1, f)),
                  const((1, f))],
        out_specs=row_spec(f),
        out_shape=jax.ShapeDtypeStruct((n, f), f32),
        scratch_shapes=[pltpu.VMEM((2, f), f32)],
    )(x, h, stats, bngr, bnbr)

    return out


# speedup vs baseline: 2.5717x; 2.4464x over previous
import functools

import jax
import jax.numpy as jnp
from jax.experimental import pallas as pl
from jax.experimental.pallas import tpu as pltpu

NUM_SEGMENTS = 512
BN_EPS = 1e-5
WN = 128
TAB = NUM_SEGMENTS + WN
_HI = jax.lax.Precision.HIGHEST
_F32 = jnp.float32
_BF = jnp.bfloat16


def _dot(a, b, prec):
    return jax.lax.dot_general(a, b, (((1,), (0,)), ((), ())),
                               precision=prec, preferred_element_type=_F32)


def _split(t):
    hi = t.astype(_BF)
    lo = (t - hi.astype(_F32)).astype(_BF)
    return hi, lo


def _oseg(o, t):
    th, tl = _split(t)
    dn = (((1,), (0,)), ((), ()))
    return (jax.lax.dot_general(o, th, dn, preferred_element_type=_F32)
            + jax.lax.dot_general(o, tl, dn, preferred_element_type=_F32))


def _ogather(o, t):
    th, tl = _split(t)
    dn = (((0,), (0,)), ((), ()))
    return (jax.lax.dot_general(o, th, dn, preferred_element_type=_F32)
            + jax.lax.dot_general(o, tl, dn, preferred_element_type=_F32))


def _onehot_narrow(b_row, s_lo, n_rows):
    ids = jax.lax.broadcasted_iota(jnp.int32, (WN, n_rows), 0)
    return (ids == (b_row - s_lo)).astype(_BF)


def _onehot_wide(b_row, n_rows):
    ids = jax.lax.broadcasted_iota(jnp.int32, (TAB, n_rows), 0)
    return (ids == b_row).astype(_BF)


def _pass_a(meta_ref, pers_ref, b3_ref, w1_ref, b1_ref, seg1_ref, acc_ref):
    i = pl.program_id(0)
    nb = pl.num_programs(0)
    br = pers_ref.shape[0]

    @pl.when(i == 0)
    def _():
        acc_ref[...] = jnp.zeros_like(acc_ref)

    x0 = jnp.maximum(_dot(pers_ref[...], w1_ref[...], _HI) + b1_ref[...], 0.0)
    x0e = jnp.concatenate([x0, jnp.ones((br, 1), _F32)], axis=1)
    b = b3_ref[0]
    s_lo = meta_ref[i, 0]
    rng = meta_ref[i, 1] - s_lo

    @pl.when(rng < WN)
    def _():
        o = _onehot_narrow(b, s_lo, br)
        acc_ref[pl.ds(s_lo, WN), :] += _oseg(o, x0e)

    @pl.when(rng >= WN)
    def _():
        o = _onehot_wide(b, br)
        acc_ref[...] += _oseg(o, x0e)

    @pl.when(i == nb - 1)
    def _():
        seg1_ref[...] = acc_ref[0:NUM_SEGMENTS, :]


def _pass_b(meta_ref, pers_ref, b3_ref, seg1_ref, w1_ref, b1_ref, g1w_ref,
            g1b_ref, l1w_ref, h1_ref, seg2_ref, c1_ref, acc_ref):
    i = pl.program_id(0)
    nb = pl.num_programs(0)
    br = pers_ref.shape[0]
    d = l1w_ref.shape[1]

    @pl.when(i == 0)
    def _():
        s = seg1_ref[...]
        cnt = jnp.clip(s[:, d:d + 1], 1.0, None)
        c1_ref[...] = jnp.zeros_like(c1_ref)
        c1_ref[0:NUM_SEGMENTS, :] = _dot(s[:, :d] / cnt, l1w_ref[...], _HI)
        acc_ref[...] = jnp.zeros_like(acc_ref)

    x0 = jnp.maximum(_dot(pers_ref[...], w1_ref[...], _HI) + b1_ref[...], 0.0)
    hg = _dot(x0, g1w_ref[...], _HI) + g1b_ref[...]
    b = b3_ref[0]
    s_lo = meta_ref[i, 0]
    rng = meta_ref[i, 1] - s_lo

    @pl.when(rng < WN)
    def _():
        o = _onehot_narrow(b, s_lo, br)
        h1 = jnp.maximum(hg - _ogather(o, c1_ref[pl.ds(s_lo, WN), :]), 0.0)
        h1_ref[...] = h1
        acc_ref[pl.ds(s_lo, WN), :] += _oseg(o, h1)

    @pl.when(rng >= WN)
    def _():
        o = _onehot_wide(b, br)
        h1 = jnp.maximum(hg - _ogather(o, c1_ref[...]), 0.0)
        h1_ref[...] = h1
        acc_ref[...] += _oseg(o, h1)

    @pl.when(i == nb - 1)
    def _():
        seg2_ref[...] = acc_ref[0:NUM_SEGMENTS, :]


def _pass_c(meta_ref, h1_ref, b3_ref, seg1_ref, seg2_ref, g2w_ref, g2b_ref,
            l2w_ref, h_ref, stats_ref, c2_ref):
    i = pl.program_id(0)
    br = h1_ref.shape[0]
    d = l2w_ref.shape[0]

    @pl.when(i == 0)
    def _():
        cnt = jnp.clip(seg1_ref[:, d:d + 1], 1.0, None)
        c2_ref[...] = jnp.zeros_like(c2_ref)
        c2_ref[0:NUM_SEGMENTS, :] = _dot(seg2_ref[...] / cnt, l2w_ref[...], _HI)
        stats_ref[...] = jnp.zeros_like(stats_ref)

    hg = _dot(h1_ref[...], g2w_ref[...], _HI) + g2b_ref[...]
    b = b3_ref[0]
    s_lo = meta_ref[i, 0]
    rng = meta_ref[i, 1] - s_lo

    @pl.when(rng < WN)
    def _():
        o = _onehot_narrow(b, s_lo, br)
        h = jnp.maximum(hg - _ogather(o, c2_ref[pl.ds(s_lo, WN), :]), 0.0)
        h_ref[...] = h
        s1 = jnp.sum(h, axis=0, keepdims=True)
        s2 = jnp.sum(h * h, axis=0, keepdims=True)
        stats_ref[...] += jnp.concatenate([s1, s2], axis=0)

    @pl.when(rng >= WN)
    def _():
        o = _onehot_wide(b, br)
        h = jnp.maximum(hg - _ogather(o, c2_ref[...]), 0.0)
        h_ref[...] = h
        s1 = jnp.sum(h, axis=0, keepdims=True)
        s2 = jnp.sum(h * h, axis=0, keepdims=True)
        stats_ref[...] += jnp.concatenate([s1, s2], axis=0)


def _pass_d(x_ref, h_ref, stats_ref, bng_ref, bnb_ref, out_ref, ss_ref, *,
            n_rows):
    i = pl.program_id(0)

    @pl.when(i == 0)
    def _():
        mu = stats_ref[0:1] * (1.0 / n_rows)
        ex2 = stats_ref[1:2] * (1.0 / n_rows)
        var = jnp.maximum(ex2 - mu * mu, 0.0)
        scale = jax.lax.rsqrt(var + BN_EPS) * bng_ref[...]
        shift = bnb_ref[...] - mu * scale
        ss_ref[...] = jnp.concatenate([scale, shift], axis=0)

    out_ref[...] = x_ref[...] + h_ref[...] * ss_ref[0:1] + ss_ref[1:2]


def kernel(x, pers, batch, W1, b1, G1W, G1b, L1W, G2W, G2b, L2W, bn_g, bn_b):
    n, f = x.shape
    p = pers.shape[1]
    d = W1.shape[1]
    br = 2000 if n % 2000 == 0 else n
    nb = n // br
    batch = batch.astype(jnp.int32)
    b3 = batch.reshape(nb, 1, br)
    meta = jnp.stack([batch[::br], batch[br - 1::br]], axis=1)
    b1r, g1br, g2br = b1.reshape(1, d), G1b.reshape(1, d), G2b.reshape(1, f)
    bngr, bnbr = bn_g.reshape(1, f), bn_b.reshape(1, f)

    row_spec = lambda w: pl.BlockSpec((br, w), lambda i: (i, 0))
    b3_spec = pl.BlockSpec((1, 1, br), lambda i: (i, 0, 0))
    const = lambda shape: pl.BlockSpec(shape, lambda i: tuple(0 for _ in shape))
    smem = pl.BlockSpec(memory_space=pltpu.SMEM)
    f32 = _F32

    seg1 = pl.pallas_call(
        _pass_a,
        grid=(nb,),
        in_specs=[smem, row_spec(p), b3_spec, const((p, d)), const((1, d))],
        out_specs=const((NUM_SEGMENTS, d + 1)),
        out_shape=jax.ShapeDtypeStruct((NUM_SEGMENTS, d + 1), f32),
        scratch_shapes=[pltpu.VMEM((TAB, d + 1), f32)],
    )(meta, pers, b3, W1, b1r)

    h1, seg2 = pl.pallas_call(
        _pass_b,
        grid=(nb,),
        in_specs=[smem, row_spec(p), b3_spec, const((NUM_SEGMENTS, d + 1)),
                  const((p, d)), const((1, d)), const((d, d)), const((1, d)),
                  const((d, d))],
        out_specs=[row_spec(d), const((NUM_SEGMENTS, d))],
        out_shape=[jax.ShapeDtypeStruct((n, d), f32),
                   jax.ShapeDtypeStruct((NUM_SEGMENTS, d), f32)],
        scratch_shapes=[pltpu.VMEM((TAB, d), f32), pltpu.VMEM((TAB, d), f32)],
    )(meta, pers, b3, seg1, W1, b1r, G1W, g1br, L1W)

    h, stats = pl.pallas_call(
        _pass_c,
        grid=(nb,),
        in_specs=[smem, row_spec(d), b3_spec, const((NUM_SEGMENTS, d + 1)),
                  const((NUM_SEGMENTS, d)), const((d, f)), const((1, f)),
                  const((d, f))],
        out_specs=[row_spec(f), const((2, f))],
        out_shape=[jax.ShapeDtypeStruct((n, f), f32),
                   jax.ShapeDtypeStruct((2, f), f32)],
        scratch_shapes=[pltpu.VMEM((TAB, f), f32)],
    )(meta, h1, b3, seg1, seg2, G2W, g2br, L2W)

    out = pl.pallas_call(
        functools.partial(_pass_d, n_rows=n),
        grid=(nb,),
        in_specs=[row_spec(f), row_spec(f), const((2, f)), const((1, f)),
                  const((1, f))],
        out_specs=row_spec(f),
        out_shape=jax.ShapeDtypeStruct((n, f), f32),
        scratch_shapes=[pltpu.VMEM((2, f), f32)],
    )(x, h, stats, bngr, bnbr)

    return out


# speedup vs baseline: 3.1362x; 1.2195x over previous
import functools

import jax
import jax.numpy as jnp
from jax.experimental import pallas as pl
from jax.experimental.pallas import tpu as pltpu

NUM_SEGMENTS = 512
BN_EPS = 1e-5
WN = 128
TAB = NUM_SEGMENTS + WN
_HI = jax.lax.Precision.HIGHEST
_F32 = jnp.float32
_BF = jnp.bfloat16


def _dot(a, b, prec):
    return jax.lax.dot_general(a, b, (((1,), (0,)), ((), ())),
                               precision=prec, preferred_element_type=_F32)


def _split(t):
    hi = t.astype(_BF)
    lo = (t - hi.astype(_F32)).astype(_BF)
    return hi, lo


def _oseg(o, t):
    th, tl = _split(t)
    dn = (((1,), (0,)), ((), ()))
    return (jax.lax.dot_general(o, th, dn, preferred_element_type=_F32)
            + jax.lax.dot_general(o, tl, dn, preferred_element_type=_F32))


def _ogather(o, t):
    th, tl = _split(t)
    dn = (((0,), (0,)), ((), ()))
    return (jax.lax.dot_general(o, th, dn, preferred_element_type=_F32)
            + jax.lax.dot_general(o, tl, dn, preferred_element_type=_F32))


def _bdot(a_bf, t):
    th, tl = _split(t)
    dn = (((1,), (0,)), ((), ()))
    return (jax.lax.dot_general(a_bf, th, dn, preferred_element_type=_F32)
            + jax.lax.dot_general(a_bf, tl, dn, preferred_element_type=_F32))


def _onehot_narrow(b_row, s_lo, n_rows):
    ids = jax.lax.broadcasted_iota(jnp.int32, (WN, n_rows), 0)
    return (ids == (b_row - s_lo)).astype(_BF)


def _onehot_wide(b_row, n_rows):
    ids = jax.lax.broadcasted_iota(jnp.int32, (TAB, n_rows), 0)
    return (ids == b_row).astype(_BF)


def _pass_a(meta_ref, pers_ref, b3_ref, w1_ref, b1_ref, seg1_ref, acc_ref):
    i = pl.program_id(0)
    nb = pl.num_programs(0)
    br = pers_ref.shape[0]

    @pl.when(i == 0)
    def _():
        acc_ref[...] = jnp.zeros_like(acc_ref)

    x0 = jnp.maximum(_dot(pers_ref[...], w1_ref[...], _HI) + b1_ref[...], 0.0)
    x0e = jnp.concatenate([x0, jnp.ones((br, 1), _F32)], axis=1)
    b = b3_ref[0]
    s_lo = meta_ref[i, 0]
    rng = meta_ref[i, 1] - s_lo

    @pl.when(rng < WN)
    def _():
        o = _onehot_narrow(b, s_lo, br)
        acc_ref[pl.ds(s_lo, WN), :] += _oseg(o, x0e)

    @pl.when(rng >= WN)
    def _():
        o = _onehot_wide(b, br)
        acc_ref[...] += _oseg(o, x0e)

    @pl.when(i == nb - 1)
    def _():
        seg1_ref[...] = acc_ref[0:NUM_SEGMENTS, :]


def _pass_b(meta_ref, pers_ref, b3_ref, seg1_ref, w1_ref, b1_ref, g1w_ref,
            g1b_ref, l1w_ref, h1_ref, seg2_ref, c1_ref, acc_ref):
    i = pl.program_id(0)
    nb = pl.num_programs(0)
    br = pers_ref.shape[0]
    d = l1w_ref.shape[1]

    @pl.when(i == 0)
    def _():
        s = seg1_ref[...]
        cnt = jnp.clip(s[:, d:d + 1], 1.0, None)
        c1_ref[...] = jnp.zeros_like(c1_ref)
        c1_ref[0:NUM_SEGMENTS, :] = _dot(s[:, :d] / cnt, l1w_ref[...], _HI)
        acc_ref[...] = jnp.zeros_like(acc_ref)

    x0 = jnp.maximum(_dot(pers_ref[...], w1_ref[...], _HI) + b1_ref[...], 0.0)
    hg = _dot(x0, g1w_ref[...], _HI) + g1b_ref[...]
    b = b3_ref[0]
    s_lo = meta_ref[i, 0]
    rng = meta_ref[i, 1] - s_lo

    @pl.when(rng < WN)
    def _():
        o = _onehot_narrow(b, s_lo, br)
        h1 = jnp.maximum(hg - _ogather(o, c1_ref[pl.ds(s_lo, WN), :]), 0.0)
        h1_ref[...] = h1.astype(_BF)
        acc_ref[pl.ds(s_lo, WN), :] += _oseg(o, h1)

    @pl.when(rng >= WN)
    def _():
        o = _onehot_wide(b, br)
        h1 = jnp.maximum(hg - _ogather(o, c1_ref[...]), 0.0)
        h1_ref[...] = h1.astype(_BF)
        acc_ref[...] += _oseg(o, h1)

    @pl.when(i == nb - 1)
    def _():
        seg2_ref[...] = acc_ref[0:NUM_SEGMENTS, :]


def _fill_c2(seg1_ref, seg2_ref, l2w_ref, c2_ref):
    d = l2w_ref.shape[0]
    cnt = jnp.clip(seg1_ref[:, d:d + 1], 1.0, None)
    c2_ref[...] = jnp.zeros_like(c2_ref)
    c2_ref[0:NUM_SEGMENTS, :] = _dot(seg2_ref[...] / cnt, l2w_ref[...], _HI)


def _pass_c(meta_ref, h1_ref, b3_ref, seg1_ref, seg2_ref, g2w_ref, g2b_ref,
            l2w_ref, stats_ref, c2_ref):
    i = pl.program_id(0)
    br = h1_ref.shape[0]

    @pl.when(i == 0)
    def _():
        _fill_c2(seg1_ref, seg2_ref, l2w_ref, c2_ref)
        stats_ref[...] = jnp.zeros_like(stats_ref)

    hg = _bdot(h1_ref[...], g2w_ref[...]) + g2b_ref[...]
    b = b3_ref[0]
    s_lo = meta_ref[i, 0]
    rng = meta_ref[i, 1] - s_lo

    @pl.when(rng < WN)
    def _():
        o = _onehot_narrow(b, s_lo, br)
        h = jnp.maximum(hg - _ogather(o, c2_ref[pl.ds(s_lo, WN), :]), 0.0)
        s1 = jnp.sum(h, axis=0, keepdims=True)
        s2 = jnp.sum(h * h, axis=0, keepdims=True)
        stats_ref[...] += jnp.concatenate([s1, s2], axis=0)

    @pl.when(rng >= WN)
    def _():
        o = _onehot_wide(b, br)
        h = jnp.maximum(hg - _ogather(o, c2_ref[...]), 0.0)
        s1 = jnp.sum(h, axis=0, keepdims=True)
        s2 = jnp.sum(h * h, axis=0, keepdims=True)
        stats_ref[...] += jnp.concatenate([s1, s2], axis=0)


def _pass_d(meta_ref, x_ref, h1_ref, b3_ref, seg1_ref, seg2_ref, g2w_ref,
            g2b_ref, l2w_ref, stats_ref, bng_ref, bnb_ref, out_ref, c2_ref,
            ss_ref, *, n_rows):
    i = pl.program_id(0)
    br = h1_ref.shape[0]

    @pl.when(i == 0)
    def _():
        _fill_c2(seg1_ref, seg2_ref, l2w_ref, c2_ref)
        mu = stats_ref[0:1] * (1.0 / n_rows)
        ex2 = stats_ref[1:2] * (1.0 / n_rows)
        var = jnp.maximum(ex2 - mu * mu, 0.0)
        scale = jax.lax.rsqrt(var + BN_EPS) * bng_ref[...]
        shift = bnb_ref[...] - mu * scale
        ss_ref[...] = jnp.concatenate([scale, shift], axis=0)

    hg = _bdot(h1_ref[...], g2w_ref[...]) + g2b_ref[...]
    b = b3_ref[0]
    s_lo = meta_ref[i, 0]
    rng = meta_ref[i, 1] - s_lo

    @pl.when(rng < WN)
    def _():
        o = _onehot_narrow(b, s_lo, br)
        h = jnp.maximum(hg - _ogather(o, c2_ref[pl.ds(s_lo, WN), :]), 0.0)
        out_ref[...] = x_ref[...] + h * ss_ref[0:1] + ss_ref[1:2]

    @pl.when(rng >= WN)
    def _():
        o = _onehot_wide(b, br)
        h = jnp.maximum(hg - _ogather(o, c2_ref[...]), 0.0)
        out_ref[...] = x_ref[...] + h * ss_ref[0:1] + ss_ref[1:2]


def kernel(x, pers, batch, W1, b1, G1W, G1b, L1W, G2W, G2b, L2W, bn_g, bn_b):
    n, f = x.shape
    p = pers.shape[1]
    d = W1.shape[1]
    br = 4000 if n % 4000 == 0 else n
    nb = n // br
    batch = batch.astype(jnp.int32)
    b3 = batch.reshape(nb, 1, br)
    meta = jnp.stack([batch[::br], batch[br - 1::br]], axis=1)
    b1r, g1br, g2br = b1.reshape(1, d), G1b.reshape(1, d), G2b.reshape(1, f)
    bngr, bnbr = bn_g.reshape(1, f), bn_b.reshape(1, f)

    row_spec = lambda w: pl.BlockSpec((br, w), lambda i: (i, 0))
    b3_spec = pl.BlockSpec((1, 1, br), lambda i: (i, 0, 0))
    const = lambda shape: pl.BlockSpec(shape, lambda i: tuple(0 for _ in shape))
    smem = pl.BlockSpec(memory_space=pltpu.SMEM)
    f32 = _F32

    seg1 = pl.pallas_call(
        _pass_a,
        grid=(nb,),
        in_specs=[smem, row_spec(p), b3_spec, const((p, d)), const((1, d))],
        out_specs=const((NUM_SEGMENTS, d + 1)),
        out_shape=jax.ShapeDtypeStruct((NUM_SEGMENTS, d + 1), f32),
        scratch_shapes=[pltpu.VMEM((TAB, d + 1), f32)],
    )(meta, pers, b3, W1, b1r)

    h1, seg2 = pl.pallas_call(
        _pass_b,
        grid=(nb,),
        in_specs=[smem, row_spec(p), b3_spec, const((NUM_SEGMENTS, d + 1)),
                  const((p, d)), const((1, d)), const((d, d)), const((1, d)),
                  const((d, d))],
        out_specs=[row_spec(d), const((NUM_SEGMENTS, d))],
        out_shape=[jax.ShapeDtypeStruct((n, d), jnp.bfloat16),
                   jax.ShapeDtypeStruct((NUM_SEGMENTS, d), f32)],
        scratch_shapes=[pltpu.VMEM((TAB, d), f32), pltpu.VMEM((TAB, d), f32)],
    )(meta, pers, b3, seg1, W1, b1r, G1W, g1br, L1W)

    stats = pl.pallas_call(
        _pass_c,
        grid=(nb,),
        in_specs=[smem, row_spec(d), b3_spec, const((NUM_SEGMENTS, d + 1)),
                  const((NUM_SEGMENTS, d)), const((d, f)), const((1, f)),
                  const((d, f))],
        out_specs=const((2, f)),
        out_shape=jax.ShapeDtypeStruct((2, f), f32),
        scratch_shapes=[pltpu.VMEM((TAB, f), f32)],
    )(meta, h1, b3, seg1, seg2, G2W, g2br, L2W)

    out = pl.pallas_call(
        functools.partial(_pass_d, n_rows=n),
        grid=(nb,),
        in_specs=[smem, row_spec(f), row_spec(d), b3_spec,
                  const((NUM_SEGMENTS, d + 1)), const((NUM_SEGMENTS, d)),
                  const((d, f)), const((1, f)), const((d, f)), const((2, f)),
                  const((1, f)), const((1, f))],
        out_specs=row_spec(f),
        out_shape=jax.ShapeDtypeStruct((n, f), f32),
        scratch_shapes=[pltpu.VMEM((TAB, f), f32), pltpu.VMEM((2, f), f32)],
    )(meta, x, h1, b3, seg1, seg2, G2W, g2br, L2W, stats, bngr, bnbr)

    return out


# speedup vs baseline: 3.3038x; 1.0534x over previous
import functools

import jax
import jax.numpy as jnp
from jax.experimental import pallas as pl
from jax.experimental.pallas import tpu as pltpu

NUM_SEGMENTS = 512
BN_EPS = 1e-5
WN = 128
TAB = NUM_SEGMENTS + WN
_HI = jax.lax.Precision.HIGHEST
_F32 = jnp.float32
_BF = jnp.bfloat16


def _dot(a, b, prec):
    return jax.lax.dot_general(a, b, (((1,), (0,)), ((), ())),
                               precision=prec, preferred_element_type=_F32)


def _split(t):
    hi = t.astype(_BF)
    lo = (t - hi.astype(_F32)).astype(_BF)
    return hi, lo


def _oseg(o, t):
    th, tl = _split(t)
    dn = (((1,), (0,)), ((), ()))
    return (jax.lax.dot_general(o, th, dn, preferred_element_type=_F32)
            + jax.lax.dot_general(o, tl, dn, preferred_element_type=_F32))


def _ogather(o, t_bf):
    dn = (((0,), (0,)), ((), ()))
    return jax.lax.dot_general(o, t_bf, dn, preferred_element_type=_F32)


def _bdot(a_bf, t):
    th, tl = _split(t)
    dn = (((1,), (0,)), ((), ()))
    return (jax.lax.dot_general(a_bf, th, dn, preferred_element_type=_F32)
            + jax.lax.dot_general(a_bf, tl, dn, preferred_element_type=_F32))


def _onehot_narrow(b_row, s_lo, n_rows):
    ids = jax.lax.broadcasted_iota(jnp.int32, (WN, n_rows), 0)
    return (ids == (b_row - s_lo)).astype(_BF)


def _onehot_wide(b_row, n_rows):
    ids = jax.lax.broadcasted_iota(jnp.int32, (TAB, n_rows), 0)
    return (ids == b_row).astype(_BF)


def _pass_a(meta_ref, pers_ref, b3_ref, w1_ref, b1_ref, seg1_ref, acc_ref):
    i = pl.program_id(0)
    nb = pl.num_programs(0)
    br = pers_ref.shape[0]

    @pl.when(i == 0)
    def _():
        acc_ref[...] = jnp.zeros_like(acc_ref)

    x0 = jnp.maximum(_dot(pers_ref[...], w1_ref[...], _HI) + b1_ref[...], 0.0)
    x0e = jnp.concatenate([x0, jnp.ones((br, 1), _F32)], axis=1)
    b = b3_ref[0]
    s_lo = meta_ref[i, 0] * 16
    rng = meta_ref[i, 1] - s_lo

    @pl.when(rng < WN)
    def _():
        o = _onehot_narrow(b, s_lo, br)
        acc_ref[pl.ds(s_lo, WN), :] += _oseg(o, x0e)

    @pl.when(rng >= WN)
    def _():
        o = _onehot_wide(b, br)
        acc_ref[...] += _oseg(o, x0e)

    @pl.when(i == nb - 1)
    def _():
        seg1_ref[...] = acc_ref[0:NUM_SEGMENTS, :]


def _pass_b(meta_ref, pers_ref, b3_ref, seg1_ref, w1_ref, b1_ref, g1w_ref,
            g1b_ref, l1w_ref, h1_ref, seg2_ref, c1_ref, acc_ref):
    i = pl.program_id(0)
    nb = pl.num_programs(0)
    br = pers_ref.shape[0]
    d = l1w_ref.shape[1]

    @pl.when(i == 0)
    def _():
        s = seg1_ref[...]
        cnt = jnp.clip(s[:, d:d + 1], 1.0, None)
        c1_ref[...] = jnp.zeros_like(c1_ref)
        c1_ref[0:NUM_SEGMENTS, :] = _dot(s[:, :d] / cnt, l1w_ref[...],
                                         _HI).astype(_BF)
        acc_ref[...] = jnp.zeros_like(acc_ref)

    x0 = jnp.maximum(_dot(pers_ref[...], w1_ref[...], _HI) + b1_ref[...], 0.0)
    hg = _dot(x0, g1w_ref[...], _HI) + g1b_ref[...]
    b = b3_ref[0]
    s_lo = meta_ref[i, 0] * 16
    rng = meta_ref[i, 1] - s_lo

    @pl.when(rng < WN)
    def _():
        o = _onehot_narrow(b, s_lo, br)
        h1 = jnp.maximum(hg - _ogather(o, c1_ref[pl.ds(s_lo, WN), :]), 0.0)
        h1_ref[...] = h1.astype(_BF)
        acc_ref[pl.ds(s_lo, WN), :] += _oseg(o, h1)

    @pl.when(rng >= WN)
    def _():
        o = _onehot_wide(b, br)
        h1 = jnp.maximum(hg - _ogather(o, c1_ref[...]), 0.0)
        h1_ref[...] = h1.astype(_BF)
        acc_ref[...] += _oseg(o, h1)

    @pl.when(i == nb - 1)
    def _():
        seg2_ref[...] = acc_ref[0:NUM_SEGMENTS, :]


def _fill_c2(seg1_ref, seg2_ref, l2w_ref, c2_ref):
    d = l2w_ref.shape[0]
    cnt = jnp.clip(seg1_ref[:, d:d + 1], 1.0, None)
    c2_ref[...] = jnp.zeros_like(c2_ref)
    c2_ref[0:NUM_SEGMENTS, :] = _dot(seg2_ref[...] / cnt, l2w_ref[...],
                                     _HI).astype(_BF)


def _pass_c(meta_ref, h1_ref, b3_ref, seg1_ref, seg2_ref, g2w_ref, g2b_ref,
            l2w_ref, stats_ref, c2_ref):
    i = pl.program_id(0)
    br = h1_ref.shape[0]

    @pl.when(i == 0)
    def _():
        _fill_c2(seg1_ref, seg2_ref, l2w_ref, c2_ref)
        stats_ref[...] = jnp.zeros_like(stats_ref)

    hg = _bdot(h1_ref[...], g2w_ref[...]) + g2b_ref[...]
    b = b3_ref[0]
    s_lo = meta_ref[i, 0] * 16
    rng = meta_ref[i, 1] - s_lo

    @pl.when(rng < WN)
    def _():
        o = _onehot_narrow(b, s_lo, br)
        h = jnp.maximum(hg - _ogather(o, c2_ref[pl.ds(s_lo, WN), :]), 0.0)
        s1 = jnp.sum(h, axis=0, keepdims=True)
        s2 = jnp.sum(h * h, axis=0, keepdims=True)
        stats_ref[...] += jnp.concatenate([s1, s2], axis=0)

    @pl.when(rng >= WN)
    def _():
        o = _onehot_wide(b, br)
        h = jnp.maximum(hg - _ogather(o, c2_ref[...]), 0.0)
        s1 = jnp.sum(h, axis=0, keepdims=True)
        s2 = jnp.sum(h * h, axis=0, keepdims=True)
        stats_ref[...] += jnp.concatenate([s1, s2], axis=0)


def _pass_d(meta_ref, x_ref, h1_ref, b3_ref, seg1_ref, seg2_ref, g2w_ref,
            g2b_ref, l2w_ref, stats_ref, bng_ref, bnb_ref, out_ref, c2_ref,
            ss_ref, *, n_rows):
    i = pl.program_id(0)
    br = h1_ref.shape[0]

    @pl.when(i == 0)
    def _():
        _fill_c2(seg1_ref, seg2_ref, l2w_ref, c2_ref)
        mu = stats_ref[0:1] * (1.0 / n_rows)
        ex2 = stats_ref[1:2] * (1.0 / n_rows)
        var = jnp.maximum(ex2 - mu * mu, 0.0)
        scale = jax.lax.rsqrt(var + BN_EPS) * bng_ref[...]
        shift = bnb_ref[...] - mu * scale
        ss_ref[...] = jnp.concatenate([scale, shift], axis=0)

    hg = _bdot(h1_ref[...], g2w_ref[...]) + g2b_ref[...]
    b = b3_ref[0]
    s_lo = meta_ref[i, 0] * 16
    rng = meta_ref[i, 1] - s_lo

    @pl.when(rng < WN)
    def _():
        o = _onehot_narrow(b, s_lo, br)
        h = jnp.maximum(hg - _ogather(o, c2_ref[pl.ds(s_lo, WN), :]), 0.0)
        out_ref[...] = x_ref[...] + h * ss_ref[0:1] + ss_ref[1:2]

    @pl.when(rng >= WN)
    def _():
        o = _onehot_wide(b, br)
        h = jnp.maximum(hg - _ogather(o, c2_ref[...]), 0.0)
        out_ref[...] = x_ref[...] + h * ss_ref[0:1] + ss_ref[1:2]


def kernel(x, pers, batch, W1, b1, G1W, G1b, L1W, G2W, G2b, L2W, bn_g, bn_b):
    n, f = x.shape
    p = pers.shape[1]
    d = W1.shape[1]
    br = 4000 if n % 4000 == 0 else n
    nb = n // br
    batch = batch.astype(jnp.int32)
    b3 = batch.reshape(nb, 1, br)
    meta = jnp.stack([batch[::br] // 16, batch[br - 1::br]], axis=1)
    b1r, g1br, g2br = b1.reshape(1, d), G1b.reshape(1, d), G2b.reshape(1, f)
    bngr, bnbr = bn_g.reshape(1, f), bn_b.reshape(1, f)

    row_spec = lambda w: pl.BlockSpec((br, w), lambda i: (i, 0))
    b3_spec = pl.BlockSpec((1, 1, br), lambda i: (i, 0, 0))
    const = lambda shape: pl.BlockSpec(shape, lambda i: tuple(0 for _ in shape))
    smem = pl.BlockSpec(memory_space=pltpu.SMEM)
    f32 = _F32

    seg1 = pl.pallas_call(
        _pass_a,
        grid=(nb,),
        in_specs=[smem, row_spec(p), b3_spec, const((p, d)), const((1, d))],
        out_specs=const((NUM_SEGMENTS, d + 1)),
        out_shape=jax.ShapeDtypeStruct((NUM_SEGMENTS, d + 1), f32),
        scratch_shapes=[pltpu.VMEM((TAB, d + 1), f32)],
    )(meta, pers, b3, W1, b1r)

    h1, seg2 = pl.pallas_call(
        _pass_b,
        grid=(nb,),
        in_specs=[smem, row_spec(p), b3_spec, const((NUM_SEGMENTS, d + 1)),
                  const((p, d)), const((1, d)), const((d, d)), const((1, d)),
                  const((d, d))],
        out_specs=[row_spec(d), const((NUM_SEGMENTS, d))],
        out_shape=[jax.ShapeDtypeStruct((n, d), jnp.bfloat16),
                   jax.ShapeDtypeStruct((NUM_SEGMENTS, d), f32)],
        scratch_shapes=[pltpu.VMEM((TAB, d), jnp.bfloat16),
                        pltpu.VMEM((TAB, d), f32)],
    )(meta, pers, b3, seg1, W1, b1r, G1W, g1br, L1W)

    stats = pl.pallas_call(
        _pass_c,
        grid=(nb,),
        in_specs=[smem, row_spec(d), b3_spec, const((NUM_SEGMENTS, d + 1)),
                  const((NUM_SEGMENTS, d)), const((d, f)), const((1, f)),
                  const((d, f))],
        out_specs=const((2, f)),
        out_shape=jax.ShapeDtypeStruct((2, f), f32),
        scratch_shapes=[pltpu.VMEM((TAB, f), jnp.bfloat16)],
    )(meta, h1, b3, seg1, seg2, G2W, g2br, L2W)

    out = pl.pallas_call(
        functools.partial(_pass_d, n_rows=n),
        grid=(nb,),
        in_specs=[smem, row_spec(f), row_spec(d), b3_spec,
                  const((NUM_SEGMENTS, d + 1)), const((NUM_SEGMENTS, d)),
                  const((d, f)), const((1, f)), const((d, f)), const((2, f)),
                  const((1, f)), const((1, f))],
        out_specs=row_spec(f),
        out_shape=jax.ShapeDtypeStruct((n, f), f32),
        scratch_shapes=[pltpu.VMEM((TAB, f), jnp.bfloat16),
                        pltpu.VMEM((2, f), f32)],
    )(meta, x, h1, b3, seg1, seg2, G2W, g2br, L2W, stats, bngr, bnbr)

    return out
